# initial kernel scaffold (unmeasured)
import jax
import jax.numpy as jnp
from jax import lax
from jax.experimental import pallas as pl
from jax.experimental.pallas import tpu as pltpu

N_DEV = 4


def kernel(x, w_mat, scale_x, scale_w):
    m_per, k = x.shape
    _, n = w_mat.shape

    x8 = x.astype(jnp.float8_e4m3fn)
    w8 = w_mat.astype(jnp.float8_e5m2)

    def body(x_ref, w_ref, sx_ref, sw_ref, out_ref, comm_ref, send_sems, recv_sems):
        my = lax.axis_index("i")
        left = (my - 1) % N_DEV
        right = (my + 1) % N_DEV

        barrier_sem = pltpu.get_barrier_semaphore()
        for nbr in (left, right):
            pl.semaphore_signal(
                barrier_sem, inc=1,
                device_id=(nbr,), device_id_type=pl.DeviceIdType.MESH,
            )
        pl.semaphore_wait(barrier_sem, 2)

        scale = sx_ref[0] * sw_ref[0]

        def gemm_store(chunk, origin):
            acc = lax.dot_general(
                chunk, w_ref[...],
                (((1,), (0,)), ((), ())),
                preferred_element_type=jnp.float32,
            )
            out_ref[pl.ds(origin * m_per, m_per), :] = jnp.maximum(acc * scale, 0.0)

        comm_ref[0] = x_ref[...]
        gemm_store(x_ref[...], my)

        for h in range(N_DEV - 1):
            send_slot = h % 2
            recv_slot = (h + 1) % 2
            rdma = pltpu.make_async_remote_copy(
                src_ref=comm_ref.at[send_slot],
                dst_ref=comm_ref.at[recv_slot],
                send_sem=send_sems.at[send_slot],
                recv_sem=recv_sems.at[recv_slot],
                device_id=(right,),
                device_id_type=pl.DeviceIdType.MESH,
            )
            rdma.start()
            rdma.wait()
            origin = (my - h - 1) % N_DEV
            gemm_store(comm_ref[recv_slot], origin)

    return pl.pallas_call(
        body,
        out_shape=jax.ShapeDtypeStruct((N_DEV * m_per, n), jnp.float32),
        in_specs=[
            pl.BlockSpec(memory_space=pltpu.VMEM),
            pl.BlockSpec(memory_space=pltpu.VMEM),
            pl.BlockSpec(memory_space=pltpu.SMEM),
            pl.BlockSpec(memory_space=pltpu.SMEM),
        ],
        out_specs=pl.BlockSpec(memory_space=pltpu.VMEM),
        scratch_shapes=[
            pltpu.VMEM((2, m_per, k), jnp.float8_e4m3fn),
            pltpu.SemaphoreType.DMA((2,)),
            pltpu.SemaphoreType.DMA((2,)),
        ],
        compiler_params=pltpu.CompilerParams(collective_id=0),
    )(x8, w8, scale_x, scale_w)


# baseline (device time: 242565 ns/iter reference)
import jax
import jax.numpy as jnp
from jax import lax
from jax.experimental import pallas as pl
from jax.experimental.pallas import tpu as pltpu

N_DEV = 4


def kernel(x, w_mat, scale_x, scale_w):
    m_per, k = x.shape
    _, n = w_mat.shape

    x8 = x.astype(jnp.float8_e4m3fn)
    w8 = w_mat.astype(jnp.float8_e5m2)

    def body(x_ref, w_ref, sx_ref, sw_ref, out_ref, comm_ref, send_sems, recv_sems):
        my = lax.axis_index("i")
        left = (my - 1) % N_DEV
        right = (my + 1) % N_DEV

        barrier_sem = pltpu.get_barrier_semaphore()
        for nbr in (left, right):
            pl.semaphore_signal(
                barrier_sem, inc=1,
                device_id=(nbr,), device_id_type=pl.DeviceIdType.MESH,
            )
        pl.semaphore_wait(barrier_sem, 2)

        scale = sx_ref[0] * sw_ref[0]

        def gemm_store(chunk, origin):
            acc = lax.dot_general(
                chunk, w_ref[...],
                (((1,), (0,)), ((), ())),
                preferred_element_type=jnp.float32,
            )
            out_ref[pl.ds(origin * m_per, m_per), :] = jnp.maximum(acc * scale, 0.0)

        comm_ref[0] = x_ref[...]
        gemm_store(x_ref[...], my)

        for h in range(N_DEV - 1):
            send_slot = h % 2
            recv_slot = (h + 1) % 2
            rdma = pltpu.make_async_remote_copy(
                src_ref=comm_ref.at[send_slot],
                dst_ref=comm_ref.at[recv_slot],
                send_sem=send_sems.at[send_slot],
                recv_sem=recv_sems.at[recv_slot],
                device_id=(right,),
                device_id_type=pl.DeviceIdType.MESH,
            )
            rdma.start()
            rdma.wait()
            origin = (my - h - 1) % N_DEV
            gemm_store(comm_ref[recv_slot], origin)

    return pl.pallas_call(
        body,
        out_shape=jax.ShapeDtypeStruct((N_DEV * m_per, n), jnp.float32),
        in_specs=[
            pl.BlockSpec(memory_space=pltpu.VMEM),
            pl.BlockSpec(memory_space=pltpu.VMEM),
            pl.BlockSpec(memory_space=pltpu.SMEM),
            pl.BlockSpec(memory_space=pltpu.SMEM),
        ],
        out_specs=pl.BlockSpec(memory_space=pltpu.VMEM),
        scratch_shapes=[
            pltpu.VMEM((2, m_per, k), jnp.float8_e4m3fn),
            pltpu.SemaphoreType.DMA((2,)),
            pltpu.SemaphoreType.DMA((2,)),
        ],
        compiler_params=pltpu.CompilerParams(
            collective_id=0, vmem_limit_bytes=100 * 1024 * 1024
        ),
    )(x8, w8, scale_x, scale_w)


# device time: 146514 ns/iter; 1.6556x vs baseline; 1.6556x over previous
import jax
import jax.numpy as jnp
from jax import lax
from jax.experimental import pallas as pl
from jax.experimental.pallas import tpu as pltpu

N_DEV = 4


def kernel(x, w_mat, scale_x, scale_w):
    m_per, k = x.shape
    _, n = w_mat.shape
    half = m_per // 2

    x8 = x.astype(jnp.float8_e4m3fn)
    w8 = w_mat.astype(jnp.float8_e5m2)

    def body(x_ref, w_ref, sx_ref, sw_ref, out_ref,
             cw_buf, ccw_buf, cw_send, cw_recv, ccw_send, ccw_recv):
        my = lax.axis_index("i")
        left = (my - 1) % N_DEV
        right = (my + 1) % N_DEV

        barrier_sem = pltpu.get_barrier_semaphore()
        for nbr in (left, right):
            pl.semaphore_signal(
                barrier_sem, inc=1,
                device_id=(nbr,), device_id_type=pl.DeviceIdType.MESH,
            )
        pl.semaphore_wait(barrier_sem, 2)

        scale = sx_ref[0] * sw_ref[0]

        def gemm_store(chunk, origin, half_idx, rows):
            acc = lax.dot_general(
                chunk, w_ref[...],
                (((1,), (0,)), ((), ())),
                preferred_element_type=jnp.float32,
            )
            out_ref[pl.ds(origin * m_per + half_idx * half, rows), :] = (
                jnp.maximum(acc * scale, 0.0)
            )

        rdmas = []

        cw = pltpu.make_async_remote_copy(
            src_ref=x_ref.at[pl.ds(0, half)],
            dst_ref=cw_buf.at[0],
            send_sem=cw_send.at[0], recv_sem=cw_recv.at[0],
            device_id=(right,), device_id_type=pl.DeviceIdType.MESH,
        )
        ccw = pltpu.make_async_remote_copy(
            src_ref=x_ref.at[pl.ds(half, half)],
            dst_ref=ccw_buf.at[0],
            send_sem=ccw_send.at[0], recv_sem=ccw_recv.at[0],
            device_id=(left,), device_id_type=pl.DeviceIdType.MESH,
        )
        cw.start()
        ccw.start()
        rdmas += [cw, ccw]

        gemm_store(x_ref[...], my, 0, m_per)

        cw.wait_recv()
        ccw.wait_recv()

        for h in (1, 2):
            cw = pltpu.make_async_remote_copy(
                src_ref=cw_buf.at[h - 1], dst_ref=cw_buf.at[h],
                send_sem=cw_send.at[h], recv_sem=cw_recv.at[h],
                device_id=(right,), device_id_type=pl.DeviceIdType.MESH,
            )
            ccw = pltpu.make_async_remote_copy(
                src_ref=ccw_buf.at[h - 1], dst_ref=ccw_buf.at[h],
                send_sem=ccw_send.at[h], recv_sem=ccw_recv.at[h],
                device_id=(left,), device_id_type=pl.DeviceIdType.MESH,
            )
            cw.start()
            ccw.start()
            rdmas += [cw, ccw]

            gemm_store(cw_buf[h - 1], (my - h) % N_DEV, 0, half)
            gemm_store(ccw_buf[h - 1], (my + h) % N_DEV, 1, half)

            cw.wait_recv()
            ccw.wait_recv()

        gemm_store(cw_buf[2], (my - 3) % N_DEV, 0, half)
        gemm_store(ccw_buf[2], (my + 3) % N_DEV, 1, half)

        for r in rdmas:
            r.wait_send()

    return pl.pallas_call(
        body,
        out_shape=jax.ShapeDtypeStruct((N_DEV * m_per, n), jnp.float32),
        in_specs=[
            pl.BlockSpec(memory_space=pltpu.VMEM),
            pl.BlockSpec(memory_space=pltpu.VMEM),
            pl.BlockSpec(memory_space=pltpu.SMEM),
            pl.BlockSpec(memory_space=pltpu.SMEM),
        ],
        out_specs=pl.BlockSpec(memory_space=pltpu.VMEM),
        scratch_shapes=[
            pltpu.VMEM((3, half, k), jnp.float8_e4m3fn),
            pltpu.VMEM((3, half, k), jnp.float8_e4m3fn),
            pltpu.SemaphoreType.DMA((3,)),
            pltpu.SemaphoreType.DMA((3,)),
            pltpu.SemaphoreType.DMA((3,)),
            pltpu.SemaphoreType.DMA((3,)),
        ],
        compiler_params=pltpu.CompilerParams(
            collective_id=0, vmem_limit_bytes=100 * 1024 * 1024
        ),
    )(x8, w8, scale_x, scale_w)


# device time: 128044 ns/iter; 1.8944x vs baseline; 1.1442x over previous
import jax
import jax.numpy as jnp
from jax import lax
from jax.experimental import pallas as pl
from jax.experimental.pallas import tpu as pltpu

N_DEV = 4


def kernel(x, w_mat, scale_x, scale_w):
    m_per, k = x.shape
    _, n = w_mat.shape
    half = m_per // 2
    NT = 16
    tk = k // NT

    def body(x_hbm, w_hbm, sx_ref, sw_ref, out_hbm,
             xstage, x8, wstage, w8, cw_buf, ccw_buf, ostage,
             ldsems, osems, cw_send, cw_recv, ccw_send, ccw_recv):
        my = lax.axis_index("i")
        left = (my - 1) % N_DEV
        right = (my + 1) % N_DEV

        x0 = pltpu.make_async_copy(
            x_hbm.at[pl.ds(0, half)], xstage, ldsems.at[0])
        x0.start()
        w_tile = lambda t, slot: pltpu.make_async_copy(
            w_hbm.at[pl.ds(t * tk, tk)], wstage.at[slot],
            ldsems.at[1 + slot])
        w_tile(0, 0).start()

        barrier_sem = pltpu.get_barrier_semaphore()
        for nbr in (left, right):
            pl.semaphore_signal(
                barrier_sem, inc=1,
                device_id=(nbr,), device_id_type=pl.DeviceIdType.MESH,
            )
        pl.semaphore_wait(barrier_sem, 2)

        scale = sx_ref[0] * sw_ref[0]
        rdmas = []

        x0.wait()
        x8[pl.ds(0, half)] = xstage[...].astype(jnp.float8_e4m3fn)
        cw = pltpu.make_async_remote_copy(
            src_ref=x8.at[pl.ds(0, half)], dst_ref=cw_buf.at[0],
            send_sem=cw_send.at[0], recv_sem=cw_recv.at[0],
            device_id=(right,), device_id_type=pl.DeviceIdType.MESH,
        )
        cw.start()
        rdmas.append(cw)

        x1 = pltpu.make_async_copy(
            x_hbm.at[pl.ds(half, half)], xstage, ldsems.at[0])
        x1.start()
        x1.wait()
        x8[pl.ds(half, half)] = xstage[...].astype(jnp.float8_e4m3fn)
        ccw = pltpu.make_async_remote_copy(
            src_ref=x8.at[pl.ds(half, half)], dst_ref=ccw_buf.at[0],
            send_sem=ccw_send.at[0], recv_sem=ccw_recv.at[0],
            device_id=(left,), device_id_type=pl.DeviceIdType.MESH,
        )
        ccw.start()
        rdmas.append(ccw)

        for t in range(NT):
            w_tile(t, t % 2).wait()
            if t + 1 < NT:
                w_tile(t + 1, (t + 1) % 2).start()
            w8[pl.ds(t * tk, tk)] = wstage[t % 2].astype(jnp.float8_e5m2)

        pending = [None, None]

        def gemm_half(chunk, origin, half_idx, slot):
            acc = lax.dot_general(
                chunk, w8[...],
                (((1,), (0,)), ((), ())),
                preferred_element_type=jnp.float32,
            )
            if pending[slot] is not None:
                pending[slot].wait()
            ostage[slot] = jnp.maximum(acc * scale, 0.0)
            cp = pltpu.make_async_copy(
                ostage.at[slot],
                out_hbm.at[pl.ds(origin * m_per + half_idx * half, half)],
                osems.at[slot])
            cp.start()
            pending[slot] = cp

        gemm_half(x8[pl.ds(0, half)], my, 0, 0)
        gemm_half(x8[pl.ds(half, half)], my, 1, 1)

        cw.wait_recv()
        ccw.wait_recv()

        for h in (1, 2):
            cw = pltpu.make_async_remote_copy(
                src_ref=cw_buf.at[h - 1], dst_ref=cw_buf.at[h],
                send_sem=cw_send.at[h], recv_sem=cw_recv.at[h],
                device_id=(right,), device_id_type=pl.DeviceIdType.MESH,
            )
            ccw = pltpu.make_async_remote_copy(
                src_ref=ccw_buf.at[h - 1], dst_ref=ccw_buf.at[h],
                send_sem=ccw_send.at[h], recv_sem=ccw_recv.at[h],
                device_id=(left,), device_id_type=pl.DeviceIdType.MESH,
            )
            cw.start()
            ccw.start()
            rdmas += [cw, ccw]

            gemm_half(cw_buf[h - 1], (my - h) % N_DEV, 0, 0)
            gemm_half(ccw_buf[h - 1], (my + h) % N_DEV, 1, 1)

            cw.wait_recv()
            ccw.wait_recv()

        gemm_half(cw_buf[2], (my - 3) % N_DEV, 0, 0)
        gemm_half(ccw_buf[2], (my + 3) % N_DEV, 1, 1)

        for r in rdmas:
            r.wait_send()
        for cp in pending:
            cp.wait()

    return pl.pallas_call(
        body,
        out_shape=jax.ShapeDtypeStruct((N_DEV * m_per, n), jnp.float32),
        in_specs=[
            pl.BlockSpec(memory_space=pl.ANY),
            pl.BlockSpec(memory_space=pl.ANY),
            pl.BlockSpec(memory_space=pltpu.SMEM),
            pl.BlockSpec(memory_space=pltpu.SMEM),
        ],
        out_specs=pl.BlockSpec(memory_space=pl.ANY),
        scratch_shapes=[
            pltpu.VMEM((half, k), jnp.float32),
            pltpu.VMEM((m_per, k), jnp.float8_e4m3fn),
            pltpu.VMEM((2, tk, n), jnp.float32),
            pltpu.VMEM((k, n), jnp.float8_e5m2),
            pltpu.VMEM((3, half, k), jnp.float8_e4m3fn),
            pltpu.VMEM((3, half, k), jnp.float8_e4m3fn),
            pltpu.VMEM((2, half, n), jnp.float32),
            pltpu.SemaphoreType.DMA((3,)),
            pltpu.SemaphoreType.DMA((2,)),
            pltpu.SemaphoreType.DMA((3,)),
            pltpu.SemaphoreType.DMA((3,)),
            pltpu.SemaphoreType.DMA((3,)),
            pltpu.SemaphoreType.DMA((3,)),
        ],
        compiler_params=pltpu.CompilerParams(
            collective_id=0, vmem_limit_bytes=100 * 1024 * 1024
        ),
    )(x, w_mat, scale_x, scale_w)


# device time: 105301 ns/iter; 2.3035x vs baseline; 1.2160x over previous
import jax
import jax.numpy as jnp
from jax import lax
from jax.experimental import pallas as pl
from jax.experimental.pallas import tpu as pltpu

N_DEV = 4
N_HOP = N_DEV - 1


def kernel(x, w_mat, scale_x, scale_w):
    m_per, k = x.shape
    _, n = w_mat.shape
    half = m_per // 2
    sub = half // 2
    NT = 16
    tk = k // NT

    def body(x_hbm, w_hbm, sx_ref, sw_ref, out_hbm,
             xstage, x8, wstage, w8, cw_buf, ccw_buf, ostage,
             xsems, wsems, osems, cw_send, cw_recv, ccw_send, ccw_recv):
        my = lax.axis_index("i")
        left = (my - 1) % N_DEV
        right = (my + 1) % N_DEV

        x_cp = lambda q, slot: pltpu.make_async_copy(
            x_hbm.at[pl.ds(q * sub, sub)], xstage.at[slot], xsems.at[slot])
        w_cp = lambda t, slot: pltpu.make_async_copy(
            w_hbm.at[pl.ds(t * tk, tk)], wstage.at[slot], wsems.at[slot])

        x_cp(0, 0).start()
        x_cp(2, 1).start()
        w_cp(0, 0).start()

        barrier_sem = pltpu.get_barrier_semaphore()
        for nbr in (left, right):
            pl.semaphore_signal(
                barrier_sem, inc=1,
                device_id=(nbr,), device_id_type=pl.DeviceIdType.MESH,
            )
        pl.semaphore_wait(barrier_sem, 2)

        w_cp(1, 1).start()
        scale = sx_ref[0] * sw_ref[0]
        rdmas = []

        def rdma(src_ref, buf, h, s, send_sems, recv_sems, dev):
            r = pltpu.make_async_remote_copy(
                src_ref=src_ref,
                dst_ref=buf.at[h, pl.ds(s * sub, sub)],
                send_sem=send_sems.at[h * 2 + s],
                recv_sem=recv_sems.at[h * 2 + s],
                device_id=(dev,), device_id_type=pl.DeviceIdType.MESH,
            )
            rdmas.append(r)
            return r

        def x_quarter(q, slot):
            x_cp(q, slot).wait()
            x8[pl.ds(q * sub, sub)] = xstage[slot].astype(jnp.float8_e4m3fn)

        x_quarter(0, 0)
        x_cp(1, 0).start()
        rdma(x8.at[pl.ds(0, sub)], cw_buf, 0, 0, cw_send, cw_recv,
             right).start()
        x_quarter(2, 1)
        x_cp(3, 1).start()
        rdma(x8.at[pl.ds(2 * sub, sub)], ccw_buf, 0, 0, ccw_send, ccw_recv,
             left).start()
        x_quarter(1, 0)
        rdma(x8.at[pl.ds(sub, sub)], cw_buf, 0, 1, cw_send, cw_recv,
             right).start()
        x_quarter(3, 1)
        rdma(x8.at[pl.ds(3 * sub, sub)], ccw_buf, 0, 1, ccw_send, ccw_recv,
             left).start()

        for t in range(NT):
            w_cp(t, t % 2).wait()
            w8[pl.ds(t * tk, tk)] = wstage[t % 2].astype(jnp.float8_e5m2)
            if t + 2 < NT:
                w_cp(t + 2, t % 2).start()

        pending = [None, None]

        def gemm_quarter(chunk, row, slot):
            acc = lax.dot_general(
                chunk, w8[...],
                (((1,), (0,)), ((), ())),
                preferred_element_type=jnp.float32,
            )
            if pending[slot] is not None:
                pending[slot].wait()
            ostage[slot] = jnp.maximum(acc * scale, 0.0)
            cp = pltpu.make_async_copy(
                ostage.at[slot], out_hbm.at[pl.ds(row, sub)], osems.at[slot])
            cp.start()
            pending[slot] = cp

        def gemm_cw(h, s, slot):
            origin = (my - h - 1) % N_DEV
            gemm_quarter(cw_buf[h, pl.ds(s * sub, sub)],
                         origin * m_per + s * sub, slot)

        def gemm_ccw(h, s, slot):
            origin = (my + h + 1) % N_DEV
            gemm_quarter(ccw_buf[h, pl.ds(s * sub, sub)],
                         origin * m_per + half + s * sub, slot)

        def recv_only(buf, h, s, send_sems, recv_sems, dev):
            return pltpu.make_async_remote_copy(
                src_ref=buf.at[h, pl.ds(s * sub, sub)],
                dst_ref=buf.at[h, pl.ds(s * sub, sub)],
                send_sem=send_sems.at[h * 2 + s],
                recv_sem=recv_sems.at[h * 2 + s],
                device_id=(dev,), device_id_type=pl.DeviceIdType.MESH,
            )

        def wait_and_forward(h, s):
            recv_only(cw_buf, h, s, cw_send, cw_recv, right).wait_recv()
            if h + 1 < N_HOP:
                rdma(cw_buf.at[h, pl.ds(s * sub, sub)], cw_buf, h + 1, s,
                     cw_send, cw_recv, right).start()
            recv_only(ccw_buf, h, s, ccw_send, ccw_recv, left).wait_recv()
            if h + 1 < N_HOP:
                rdma(ccw_buf.at[h, pl.ds(s * sub, sub)], ccw_buf, h + 1, s,
                     ccw_send, ccw_recv, left).start()

        wait_and_forward(0, 0)
        gemm_quarter(x8[pl.ds(0, sub)], my * m_per, 0)
        gemm_quarter(x8[pl.ds(sub, sub)], my * m_per + sub, 1)
        wait_and_forward(0, 1)
        gemm_quarter(x8[pl.ds(2 * sub, sub)], my * m_per + 2 * sub, 0)
        gemm_quarter(x8[pl.ds(3 * sub, sub)], my * m_per + 3 * sub, 1)
        gemm_cw(0, 0, 0)
        gemm_ccw(0, 0, 1)
        wait_and_forward(1, 0)
        gemm_cw(0, 1, 0)
        gemm_ccw(0, 1, 1)
        wait_and_forward(1, 1)
        gemm_cw(1, 0, 0)
        gemm_ccw(1, 0, 1)
        gemm_cw(1, 1, 0)
        gemm_ccw(1, 1, 1)
        wait_and_forward(2, 0)
        gemm_cw(2, 0, 0)
        gemm_ccw(2, 0, 1)
        wait_and_forward(2, 1)
        gemm_cw(2, 1, 0)
        gemm_ccw(2, 1, 1)

        for r in rdmas:
            r.wait_send()
        for cp in pending:
            cp.wait()

    return pl.pallas_call(
        body,
        out_shape=jax.ShapeDtypeStruct((N_DEV * m_per, n), jnp.float32),
        in_specs=[
            pl.BlockSpec(memory_space=pl.ANY),
            pl.BlockSpec(memory_space=pl.ANY),
            pl.BlockSpec(memory_space=pltpu.SMEM),
            pl.BlockSpec(memory_space=pltpu.SMEM),
        ],
        out_specs=pl.BlockSpec(memory_space=pl.ANY),
        scratch_shapes=[
            pltpu.VMEM((2, sub, k), jnp.float32),
            pltpu.VMEM((m_per, k), jnp.float8_e4m3fn),
            pltpu.VMEM((2, tk, n), jnp.float32),
            pltpu.VMEM((k, n), jnp.float8_e5m2),
            pltpu.VMEM((N_HOP, half, k), jnp.float8_e4m3fn),
            pltpu.VMEM((N_HOP, half, k), jnp.float8_e4m3fn),
            pltpu.VMEM((2, sub, n), jnp.float32),
            pltpu.SemaphoreType.DMA((2,)),
            pltpu.SemaphoreType.DMA((2,)),
            pltpu.SemaphoreType.DMA((2,)),
            pltpu.SemaphoreType.DMA((N_HOP * 2,)),
            pltpu.SemaphoreType.DMA((N_HOP * 2,)),
            pltpu.SemaphoreType.DMA((N_HOP * 2,)),
            pltpu.SemaphoreType.DMA((N_HOP * 2,)),
        ],
        compiler_params=pltpu.CompilerParams(
            collective_id=0, vmem_limit_bytes=100 * 1024 * 1024
        ),
    )(x, w_mat, scale_x, scale_w)
